# baseline (speedup 1.0000x reference)
import jax
import jax.numpy as jnp
from jax import lax
from jax.experimental import pallas as pl
from jax.experimental.pallas import tpu as pltpu


_DN = (((1,), (1,)), ((), ()))


def _round_up(a, m):
    return ((a + m - 1) // m) * m


def _make_body(bm, bn):
    def body(x_hbm, w_ref, b_ref, y_out, yref_out, xf32_ref, xb_ref, sem):
        i = pl.program_id(0)
        j = pl.program_id(1)

        @pl.when(j == 0)
        def _():
            cp = pltpu.make_async_copy(
                x_hbm.at[pl.ds(i * bm, bm), :], xf32_ref, sem)
            cp.start()
            cp.wait()
            xb_ref[...] = xf32_ref[...].astype(jnp.bfloat16)

        xb = xb_ref[...]
        b = b_ref[...]
        half = bn // 2
        for h in range(2):
            lo, hi = h * half, (h + 1) * half
            wh = w_ref[lo:hi, :].astype(jnp.bfloat16)
            acc = lax.dot_general(xb, wh, dimension_numbers=_DN,
                                  preferred_element_type=jnp.float32)
            bh = b[:, lo:hi]
            y_out[:, lo:hi] = (acc + bh).astype(y_out.dtype)
            yref_out[:, lo:hi] = (0.5 * acc + bh).astype(yref_out.dtype)

    return body


def kernel(x, weight, bias_vec):
    B, In = x.shape
    Out, In_w = weight.shape
    assert In_w == In

    bm = min(1024, _round_up(B, 8))
    bn = min(512, _round_up(Out, 256))

    Bp = _round_up(B, bm)
    Outp = _round_up(Out, bn)
    Inp = _round_up(In, 128)

    xp = x if (Bp, Inp) == (B, In) else jnp.pad(x, ((0, Bp - B), (0, Inp - In)))
    wp = (weight if (Outp, Inp) == (Out, In)
          else jnp.pad(weight, ((0, Outp - Out), (0, Inp - In))))
    bp = bias_vec if Outp == Out else jnp.pad(bias_vec, (0, Outp - Out))
    b2 = bp.reshape(1, Outp)

    grid = (Bp // bm, Outp // bn)

    y_p, yref_p = pl.pallas_call(
        _make_body(bm, bn),
        out_shape=(jax.ShapeDtypeStruct((Bp, Outp), x.dtype),
                   jax.ShapeDtypeStruct((Bp, Outp), x.dtype)),
        grid=grid,
        in_specs=[pl.BlockSpec(memory_space=pl.ANY),
                  pl.BlockSpec((bn, Inp), lambda i, j: (j, 0)),
                  pl.BlockSpec((1, bn), lambda i, j: (0, j))],
        out_specs=[pl.BlockSpec((bm, bn), lambda i, j: (i, j)),
                   pl.BlockSpec((bm, bn), lambda i, j: (i, j))],
        scratch_shapes=[pltpu.VMEM((bm, Inp), jnp.float32),
                        pltpu.VMEM((bm, Inp), jnp.bfloat16),
                        pltpu.SemaphoreType.DMA],
        compiler_params=pltpu.CompilerParams(
            dimension_semantics=("parallel", "arbitrary"),
            vmem_limit_bytes=64 * 1024 * 1024),
    )(xp, wp, b2)

    if (Bp, Outp) == (B, Out):
        return y_p, yref_p
    return y_p[:B, :Out], yref_p[:B, :Out]


# speedup vs baseline: 1.0785x; 1.0785x over previous
import jax
import jax.numpy as jnp
from jax import lax
from jax.experimental import pallas as pl
from jax.experimental.pallas import tpu as pltpu


_DN = (((1,), (1,)), ((), ()))


def _dual_out_kernel(x_ref, w_ref, b_ref, y_out, yref_out):
    xb = x_ref[...].astype(jnp.bfloat16)
    b = b_ref[...]
    bn = w_ref.shape[0]
    half = bn // 2
    for h in range(2):
        lo, hi = h * half, (h + 1) * half
        wh = w_ref[lo:hi, :].astype(jnp.bfloat16)
        acc = lax.dot_general(xb, wh, dimension_numbers=_DN,
                              preferred_element_type=jnp.float32)
        bh = b[:, lo:hi]
        y_out[:, lo:hi] = (acc + bh).astype(y_out.dtype)
        yref_out[:, lo:hi] = (0.5 * acc + bh).astype(yref_out.dtype)


def _round_up(a, m):
    return ((a + m - 1) // m) * m


def kernel(x, weight, bias_vec):
    B, In = x.shape
    Out, In_w = weight.shape
    assert In_w == In

    bm = min(1024, _round_up(B, 8))
    bn = min(512, _round_up(Out, 256))

    Bp = _round_up(B, bm)
    Outp = _round_up(Out, bn)
    Inp = _round_up(In, 128)

    xp = x if (Bp, Inp) == (B, In) else jnp.pad(x, ((0, Bp - B), (0, Inp - In)))
    wp = (weight if (Outp, Inp) == (Out, In)
          else jnp.pad(weight, ((0, Outp - Out), (0, Inp - In))))
    bp = bias_vec if Outp == Out else jnp.pad(bias_vec, (0, Outp - Out))
    b2 = bp.reshape(1, Outp)

    grid = (Bp // bm, Outp // bn)

    y_p, yref_p = pl.pallas_call(
        _dual_out_kernel,
        out_shape=(jax.ShapeDtypeStruct((Bp, Outp), x.dtype),
                   jax.ShapeDtypeStruct((Bp, Outp), x.dtype)),
        grid=grid,
        in_specs=[pl.BlockSpec((bm, Inp), lambda i, j: (i, 0)),
                  pl.BlockSpec((bn, Inp), lambda i, j: (j, 0)),
                  pl.BlockSpec((1, bn), lambda i, j: (0, j))],
        out_specs=[pl.BlockSpec((bm, bn), lambda i, j: (i, j)),
                   pl.BlockSpec((bm, bn), lambda i, j: (i, j))],
        compiler_params=pltpu.CompilerParams(
            dimension_semantics=("parallel", "parallel"),
            vmem_limit_bytes=64 * 1024 * 1024),
    )(xp, wp, b2)

    if (Bp, Outp) == (B, Out):
        return y_p, yref_p
    return y_p[:B, :Out], yref_p[:B, :Out]
